# baseline (speedup 1.0000x reference)
import functools

import jax
import jax.numpy as jnp
from jax import lax
from jax.experimental import pallas as pl
from jax.experimental.pallas import tpu as pltpu
from jax.experimental.pallas import tpu_sc as plsc

N = 10000
NE = 320000
D = 128
DH = 64
DP = 16
K = 128
NCH = NE // K
ACC_ROWS = 10112
NBUF = 5

_mesh = plsc.VectorSubcoreMesh(core_axis_name="c", subcore_axis_name="s")


NG = 5
G = 2
DL = 3
NI = 10


def _pipeline(tab, acc, ei_hbm, ch0, cpt, idx_v, rows_v, sems_i, sems_g,
              sems_s):
    def ei(c):
        return ei_hbm.at[:, pl.ds(c * K, K)]

    def gather(c, b):
        return pltpu.make_async_copy(tab.at[idx_v.at[b % NI, 0]],
                                     rows_v.at[b % NG], sems_g[b % NG])

    def scatter(c, b):
        return pltpu.make_async_copy(rows_v.at[b % NG],
                                     acc.at[idx_v.at[b % NI, 1]],
                                     sems_s[b % NG])

    for b in range(NI):
        pltpu.async_copy(ei(ch0 + b), idx_v.at[b], sems_i[b])
    for b in range(G):
        pltpu.make_async_copy(ei(ch0 + b), idx_v.at[b], sems_i[b]).wait()
        gather(b, b).start()

    def slot(j, b, tail):
        @pl.when(j < cpt) if tail else _now
        def _():
            gather(j, b).wait()
            scatter(j, b).start(add=True)

        @pl.when((j >= DL) & (j - DL < cpt))
        def _():
            scatter(j - DL, b - DL).wait()

        @pl.when((j >= DL) & (j + NI - DL < cpt))
        def _():
            pltpu.async_copy(ei(ch0 + j + NI - DL), idx_v.at[(b - DL) % NI],
                             sems_i[(b - DL) % NI])

        @pl.when(j + G < cpt)
        def _():
            pltpu.make_async_copy(ei(ch0 + j + G), idx_v.at[(b + G) % NI],
                                  sems_i[(b + G) % NI]).wait()
            gather(j + G, b + G).start()

    def step(t, carry):
        for b in range(NI):
            slot(t * NI + b, b, tail=False)
        return carry

    nfull = cpt // NI
    lax.fori_loop(0, nfull, step, 0)
    for b in range(NI + DL):
        slot(nfull * NI + b, b, tail=True)


def _now(f):
    return f()


def _segsum1_body(q_hbm, ei_hbm, zeros_hbm, out_hbm,
                  idx_v, rows_v, qbuf, acc, *sems):
    cid = lax.axis_index("c")
    sid = lax.axis_index("s")
    sems_i = sems[:NI]
    sems_g = sems[NI:NI + NG]
    sems_s = sems[NI + NG:]

    pltpu.sync_copy(q_hbm.at[pl.ds(sid * 625, 625), pl.ds(cid * DH, DH)],
                    qbuf.at[pl.ds(sid * 625, 625)])
    rpz = ACC_ROWS // 16
    pltpu.sync_copy(zeros_hbm.at[pl.ds(sid * rpz, rpz)],
                    acc.at[pl.ds(sid * rpz, rpz)])
    plsc.subcore_barrier()

    cpt = jnp.where(sid < 4, 157, 156)
    ch0 = sid * 156 + jnp.minimum(sid, 4)
    _pipeline(qbuf, acc, ei_hbm, ch0, cpt, idx_v, rows_v, sems_i,
              sems_g, sems_s)
    plsc.subcore_barrier()

    pltpu.sync_copy(acc.at[pl.ds(sid * rpz, rpz)],
                    out_hbm.at[pl.ds(sid * rpz, rpz), pl.ds(cid * DH, DH)])


def _segsum2_body(p_hbm, ei_hbm, zeros_hbm, out_hbm,
                  idx_v, rows_v, pbuf, acc, *sems):
    cid = lax.axis_index("c")
    sid = lax.axis_index("s")
    sems_i = sems[:NI]
    sems_g = sems[NI:NI + NG]
    sems_s = sems[NI + NG:]

    pltpu.sync_copy(p_hbm.at[pl.ds(sid * 625, 625)],
                    pbuf.at[pl.ds(sid * 625, 625)])
    rpz = ACC_ROWS // 16
    pltpu.sync_copy(zeros_hbm.at[pl.ds(sid * rpz, rpz), pl.ds(0, DP)],
                    acc.at[pl.ds(sid * rpz, rpz)])
    plsc.subcore_barrier()

    wid = cid * 16 + sid
    cpt = jnp.where(wid < 4, 79, 78)
    ch0 = wid * 78 + jnp.minimum(wid, 4)
    _pipeline(pbuf, acc, ei_hbm, ch0, cpt, idx_v, rows_v, sems_i,
              sems_g, sems_s)
    plsc.subcore_barrier()

    pltpu.sync_copy(acc.at[pl.ds(sid * rpz, rpz)],
                    out_hbm.at[cid].at[pl.ds(sid * rpz, rpz)])


_segsum1 = functools.partial(
    pl.kernel,
    out_type=jax.ShapeDtypeStruct((ACC_ROWS, D), jnp.float32),
    mesh=_mesh,
    scratch_types=[
        pltpu.VMEM((NI, 2, K), jnp.int32),
        pltpu.VMEM((NG, K, DH), jnp.float32),
        pltpu.VMEM_SHARED((N, DH), jnp.float32),
        pltpu.VMEM_SHARED((ACC_ROWS, DH), jnp.float32),
    ] + [pltpu.SemaphoreType.DMA] * (NI + 2 * NG),
    compiler_params=pltpu.CompilerParams(use_tc_tiling_on_sc=False),
)(_segsum1_body)

_segsum2 = functools.partial(
    pl.kernel,
    out_type=jax.ShapeDtypeStruct((2, ACC_ROWS, DP), jnp.float32),
    mesh=_mesh,
    scratch_types=[
        pltpu.VMEM((NI, 2, K), jnp.int32),
        pltpu.VMEM((NG, K, DP), jnp.float32),
        pltpu.VMEM_SHARED((N, DP), jnp.float32),
        pltpu.VMEM_SHARED((ACC_ROWS, DP), jnp.float32),
    ] + [pltpu.SemaphoreType.DMA] * (NI + 2 * NG),
    compiler_params=pltpu.CompilerParams(use_tc_tiling_on_sc=False),
)(_segsum2_body)


def _mm_body(x_ref, w_ref, o_ref):
    o_ref[...] = lax.dot_general(
        x_ref[...], w_ref[...], (((1,), (1,)), ((), ())),
        preferred_element_type=jnp.float32)


def _relu_mm_body(q_ref, s_ref, b1_ref, w2_ref, o_ref):
    h = jnp.maximum(q_ref[...] + s_ref[...] + b1_ref[...], 0.0)
    o_ref[...] = jnp.dot(h, w2_ref[...], preferred_element_type=jnp.float32)


_CROWS = ACC_ROWS // 32


def _combine_sc_body(p_hbm, s_hbm, b2_hbm, out_hbm, pv, sav, sbv, b2v, ov):
    wid = lax.axis_index("c") * 16 + lax.axis_index("s")
    r0 = wid * _CROWS
    pltpu.sync_copy(p_hbm.at[pl.ds(r0, _CROWS)], pv)
    pltpu.sync_copy(s_hbm.at[0].at[pl.ds(r0, _CROWS)], sav)
    pltpu.sync_copy(s_hbm.at[1].at[pl.ds(r0, _CROWS)], sbv)
    pltpu.sync_copy(b2_hbm, b2v)
    b2row = b2v[...]

    def row(i, carry):
        ov[i] = pv[i] + sav[i] + sbv[i] + b2row
        return carry

    lax.fori_loop(0, _CROWS, row, 0)
    pltpu.sync_copy(ov, out_hbm.at[pl.ds(r0, _CROWS)])


_combine = functools.partial(
    pl.kernel,
    out_type=jax.ShapeDtypeStruct((ACC_ROWS, DP), jnp.float32),
    mesh=_mesh,
    scratch_types=[
        pltpu.VMEM((_CROWS, DP), jnp.float32),
        pltpu.VMEM((_CROWS, DP), jnp.float32),
        pltpu.VMEM((_CROWS, DP), jnp.float32),
        pltpu.VMEM((DP,), jnp.float32),
        pltpu.VMEM((_CROWS, DP), jnp.float32),
    ],
    compiler_params=pltpu.CompilerParams(use_tc_tiling_on_sc=False),
)(_combine_sc_body)


_RB = 2000


def kernel(x, edge_index, W1, b1, W2, b2):
    ei = edge_index.astype(jnp.int32)

    zeros64 = jnp.zeros((ACC_ROWS, DH), jnp.float32)
    W2p = jnp.pad(W2.T, ((0, 0), (0, DP - 2)))

    q = pl.pallas_call(
        _mm_body,
        out_shape=jax.ShapeDtypeStruct((N, D), jnp.float32),
    )(x, W1)

    s1 = _segsum1(q, ei, zeros64)

    p16 = pl.pallas_call(
        _relu_mm_body,
        grid=(1,),
        in_specs=[pl.BlockSpec((N, D), lambda i: (0, 0)),
                  pl.BlockSpec((N, D), lambda i: (0, 0)),
                  pl.BlockSpec((1, D), lambda i: (0, 0)),
                  pl.BlockSpec((D, DP), lambda i: (0, 0))],
        out_specs=pl.BlockSpec((N, DP), lambda i: (0, 0)),
        out_shape=jax.ShapeDtypeStruct((ACC_ROWS, DP), jnp.float32),
    )(q, s1, b1[None, :], W2p)

    s2 = _segsum2(p16, ei, zeros64)

    b2p = jnp.pad(b2, (0, DP - 2))
    out16 = _combine(p16, s2, b2p)

    return out16[:N, :2]


# speedup vs baseline: 1.0077x; 1.0077x over previous
import functools

import jax
import jax.numpy as jnp
from jax import lax
from jax.experimental import pallas as pl
from jax.experimental.pallas import tpu as pltpu
from jax.experimental.pallas import tpu_sc as plsc

N = 10000
NE = 320000
D = 128
DH = 64
DP = 16
K = 128
NCH = NE // K
ACC_ROWS = 10112
_mesh = plsc.VectorSubcoreMesh(core_axis_name="c", subcore_axis_name="s")


NG = 5
G = 3
DL = 2
NI = 10


def _pipeline(tab, acc, ei_hbm, ch0, cpt, idx_v, rows_v, sems_i, sems_g,
              sems_s):
    def ei(c):
        return ei_hbm.at[:, pl.ds(c * K, K)]

    def gather(c, b):
        return pltpu.make_async_copy(tab.at[idx_v.at[b % NI, 0]],
                                     rows_v.at[b % NG], sems_g[b % NG])

    def scatter(c, b):
        return pltpu.make_async_copy(rows_v.at[b % NG],
                                     acc.at[idx_v.at[b % NI, 1]],
                                     sems_s[b % NG])

    for b in range(NI):
        pltpu.async_copy(ei(ch0 + b), idx_v.at[b], sems_i[b])
    for b in range(G):
        pltpu.make_async_copy(ei(ch0 + b), idx_v.at[b], sems_i[b]).wait()
        gather(b, b).start()

    def slot(j, b, tail):
        @pl.when(j < cpt) if tail else _now
        def _():
            gather(j, b).wait()
            scatter(j, b).start(add=True)

        @pl.when((j >= DL) & (j - DL < cpt))
        def _():
            scatter(j - DL, b - DL).wait()

        @pl.when((j >= DL) & (j + NI - DL < cpt))
        def _():
            pltpu.async_copy(ei(ch0 + j + NI - DL), idx_v.at[(b - DL) % NI],
                             sems_i[(b - DL) % NI])

        @pl.when(j + G < cpt)
        def _():
            pltpu.make_async_copy(ei(ch0 + j + G), idx_v.at[(b + G) % NI],
                                  sems_i[(b + G) % NI]).wait()
            gather(j + G, b + G).start()

    def step(t, carry):
        for b in range(NI):
            slot(t * NI + b, b, tail=False)
        return carry

    nfull = cpt // NI
    lax.fori_loop(0, nfull, step, 0)
    for b in range(NI + DL):
        slot(nfull * NI + b, b, tail=True)


def _now(f):
    return f()


def _segsum1_body(q_hbm, ei_hbm, zeros_hbm, out_hbm,
                  idx_v, rows_v, qbuf, acc, *sems):
    cid = lax.axis_index("c")
    sid = lax.axis_index("s")
    sems_i = sems[:NI]
    sems_g = sems[NI:NI + NG]
    sems_s = sems[NI + NG:]

    pltpu.sync_copy(q_hbm.at[pl.ds(sid * 625, 625), pl.ds(cid * DH, DH)],
                    qbuf.at[pl.ds(sid * 625, 625)])
    rpz = ACC_ROWS // 16
    pltpu.sync_copy(zeros_hbm.at[pl.ds(sid * rpz, rpz)],
                    acc.at[pl.ds(sid * rpz, rpz)])
    plsc.subcore_barrier()

    cpt = jnp.where(sid < 4, 157, 156)
    ch0 = sid * 156 + jnp.minimum(sid, 4)
    _pipeline(qbuf, acc, ei_hbm, ch0, cpt, idx_v, rows_v, sems_i,
              sems_g, sems_s)
    plsc.subcore_barrier()

    pltpu.sync_copy(acc.at[pl.ds(sid * rpz, rpz)],
                    out_hbm.at[pl.ds(sid * rpz, rpz), pl.ds(cid * DH, DH)])


def _segsum2_body(p_hbm, ei_hbm, zeros_hbm, out_hbm,
                  idx_v, rows_v, pbuf, acc, *sems):
    cid = lax.axis_index("c")
    sid = lax.axis_index("s")
    sems_i = sems[:NI]
    sems_g = sems[NI:NI + NG]
    sems_s = sems[NI + NG:]

    pltpu.sync_copy(p_hbm.at[pl.ds(sid * 625, 625)],
                    pbuf.at[pl.ds(sid * 625, 625)])
    rpz = ACC_ROWS // 16
    pltpu.sync_copy(zeros_hbm.at[pl.ds(sid * rpz, rpz), pl.ds(0, DP)],
                    acc.at[pl.ds(sid * rpz, rpz)])
    plsc.subcore_barrier()

    wid = cid * 16 + sid
    cpt = jnp.where(wid < 4, 79, 78)
    ch0 = wid * 78 + jnp.minimum(wid, 4)
    _pipeline(pbuf, acc, ei_hbm, ch0, cpt, idx_v, rows_v, sems_i,
              sems_g, sems_s)
    plsc.subcore_barrier()

    pltpu.sync_copy(acc.at[pl.ds(sid * rpz, rpz)],
                    out_hbm.at[cid].at[pl.ds(sid * rpz, rpz)])


_segsum1 = functools.partial(
    pl.kernel,
    out_type=jax.ShapeDtypeStruct((ACC_ROWS, D), jnp.float32),
    mesh=_mesh,
    scratch_types=[
        pltpu.VMEM((NI, 2, K), jnp.int32),
        pltpu.VMEM((NG, K, DH), jnp.float32),
        pltpu.VMEM_SHARED((N, DH), jnp.float32),
        pltpu.VMEM_SHARED((ACC_ROWS, DH), jnp.float32),
    ] + [pltpu.SemaphoreType.DMA] * (NI + 2 * NG),
    compiler_params=pltpu.CompilerParams(use_tc_tiling_on_sc=False),
)(_segsum1_body)

_segsum2 = functools.partial(
    pl.kernel,
    out_type=jax.ShapeDtypeStruct((2, ACC_ROWS, DP), jnp.float32),
    mesh=_mesh,
    scratch_types=[
        pltpu.VMEM((NI, 2, K), jnp.int32),
        pltpu.VMEM((NG, K, DP), jnp.float32),
        pltpu.VMEM_SHARED((N, DP), jnp.float32),
        pltpu.VMEM_SHARED((ACC_ROWS, DP), jnp.float32),
    ] + [pltpu.SemaphoreType.DMA] * (NI + 2 * NG),
    compiler_params=pltpu.CompilerParams(use_tc_tiling_on_sc=False),
)(_segsum2_body)


def _mm_body(x_ref, w_ref, o_ref):
    o_ref[...] = lax.dot_general(
        x_ref[...], w_ref[...], (((1,), (1,)), ((), ())),
        preferred_element_type=jnp.float32)


def _relu_mm_body(q_ref, s_ref, b1_ref, w2_ref, o_ref):
    h = jnp.maximum(q_ref[...] + s_ref[...] + b1_ref[...], 0.0)
    o_ref[...] = jnp.dot(h, w2_ref[...], preferred_element_type=jnp.float32)


_CROWS = ACC_ROWS // 32


def _combine_sc_body(p_hbm, s_hbm, b2_hbm, out_hbm, pv, sav, sbv, b2v, ov):
    wid = lax.axis_index("c") * 16 + lax.axis_index("s")
    r0 = wid * _CROWS
    pltpu.sync_copy(p_hbm.at[pl.ds(r0, _CROWS)], pv)
    pltpu.sync_copy(s_hbm.at[0].at[pl.ds(r0, _CROWS)], sav)
    pltpu.sync_copy(s_hbm.at[1].at[pl.ds(r0, _CROWS)], sbv)
    pltpu.sync_copy(b2_hbm, b2v)
    b2row = b2v[...]

    def row(i, carry):
        ov[i] = pv[i] + sav[i] + sbv[i] + b2row
        return carry

    lax.fori_loop(0, _CROWS, row, 0)
    pltpu.sync_copy(ov, out_hbm.at[pl.ds(r0, _CROWS)])


_combine = functools.partial(
    pl.kernel,
    out_type=jax.ShapeDtypeStruct((ACC_ROWS, DP), jnp.float32),
    mesh=_mesh,
    scratch_types=[
        pltpu.VMEM((_CROWS, DP), jnp.float32),
        pltpu.VMEM((_CROWS, DP), jnp.float32),
        pltpu.VMEM((_CROWS, DP), jnp.float32),
        pltpu.VMEM((DP,), jnp.float32),
        pltpu.VMEM((_CROWS, DP), jnp.float32),
    ],
    compiler_params=pltpu.CompilerParams(use_tc_tiling_on_sc=False),
)(_combine_sc_body)


_RB = 2000


def kernel(x, edge_index, W1, b1, W2, b2):
    ei = edge_index.astype(jnp.int32)

    zeros64 = jnp.zeros((ACC_ROWS, DH), jnp.float32)
    W2p = jnp.pad(W2.T, ((0, 0), (0, DP - 2)))

    q = pl.pallas_call(
        _mm_body,
        out_shape=jax.ShapeDtypeStruct((N, D), jnp.float32),
    )(x, W1)

    s1 = _segsum1(q, ei, zeros64)

    p16 = pl.pallas_call(
        _relu_mm_body,
        grid=(1,),
        in_specs=[pl.BlockSpec((N, D), lambda i: (0, 0)),
                  pl.BlockSpec((N, D), lambda i: (0, 0)),
                  pl.BlockSpec((1, D), lambda i: (0, 0)),
                  pl.BlockSpec((D, DP), lambda i: (0, 0))],
        out_specs=pl.BlockSpec((N, DP), lambda i: (0, 0)),
        out_shape=jax.ShapeDtypeStruct((ACC_ROWS, DP), jnp.float32),
    )(q, s1, b1[None, :], W2p)

    s2 = _segsum2(p16, ei, zeros64)

    b2p = jnp.pad(b2, (0, DP - 2))
    out16 = _combine(p16, s2, b2p)

    return out16[:N, :2]
